# Initial kernel scaffold; baseline (speedup 1.0000x reference)
import jax
import jax.numpy as jnp
from jax.experimental import pallas as pl


def kernel(gate_type, edge_index, edge_attr, emb_c, W1_c, W2_c, b2_c, Wc1, bc1, Wc2, bc2, emb_a, W1_a, W2_a, b2_a, Wa1, ba1, Wa2, ba2):
    raise NotImplementedError("write your pallas kernel here")


# speedup vs baseline: 1.3446x; 1.3446x over previous
import functools

import jax
import jax.numpy as jnp
from jax import lax
from jax.experimental import pallas as pl
from jax.experimental.pallas import tpu as pltpu
from jax.experimental.pallas import tpu_sc as plsc

N = 10000
E = 320000
D = 128
GATE_TYPES = 26
NUM_OUT = 64

NC = 2
NS = 16
LANES = 16

N_PAD = 10240
ROWW = 144
K = 128
CHUNKS = 157
E_PT = K * CHUNKS
E_PAD = NS * E_PT
NODES_PT = N_PAD // NS


def _prep_body(ec_ref, wc_ref, ea_ref, wa_ref, out_ref):
    out_ref[0] = jnp.dot(ec_ref[...], wc_ref[...],
                         preferred_element_type=jnp.float32)
    out_ref[1] = jnp.dot(ea_ref[...], wa_ref[...],
                         preferred_element_type=jnp.float32)


def _prep_tables(embx_c, w1f_c, embx_a, w1f_a):
    return pl.pallas_call(
        _prep_body,
        out_shape=jax.ShapeDtypeStruct((2, 32, D), jnp.float32),
    )(embx_c, w1f_c, embx_a, w1f_a)


def _edge_body(gb_hbm, gt_hbm, src_hbm, dst_hbm, attr_hbm, zeros_hbm,
               out_hbm, gb_v, gt_v, src_v, dst_v, a_v, rows_v, acc_sh):
    c = lax.axis_index("c")
    s = lax.axis_index("s")

    pltpu.sync_copy(zeros_hbm, acc_sh.at[pl.ds(s * NODES_PT, NODES_PT)])
    pltpu.sync_copy(gb_hbm.at[c], gb_v)
    pltpu.sync_copy(gt_hbm, gt_v)
    plsc.subcore_barrier()

    iota = lax.iota(jnp.int32, LANES)
    b1 = [gb_v[pl.ds(26 * D + j * LANES, LANES)] for j in range(D // LANES)]
    b2 = [gb_v[pl.ds(27 * D + j * LANES, LANES)] for j in range(D // LANES)]
    b3 = [gb_v[pl.ds(28 * D + j * LANES, LANES)] for j in range(D // LANES)]
    cnt_vec = gb_v[pl.ds(29 * D, LANES)]

    base0 = s * E_PT

    def chunk_body(i, carry):
        base = base0 + i * K
        pltpu.sync_copy(src_hbm.at[pl.ds(base, K)], src_v)
        pltpu.sync_copy(dst_hbm.at[pl.ds(base, K)], dst_v)
        pltpu.sync_copy(attr_hbm.at[pl.ds(3 * base, 3 * K)], a_v)

        def edge_body(e, carry2):
            e16 = jnp.full((LANES,), e, jnp.int32)
            s_b = plsc.load_gather(src_v, [e16])
            t_b = plsc.load_gather(gt_v, [s_b])
            a1 = plsc.load_gather(a_v, [e16])
            a2 = plsc.load_gather(a_v, [e16 + K])
            a3 = plsc.load_gather(a_v, [e16 + 2 * K])
            gbase = t_b * D + iota
            for j in range(D // LANES):
                g = plsc.load_gather(gb_v, [gbase + j * LANES])
                z = g + a1 * b1[j] + a2 * b2[j] + a3 * b3[j]
                r = jnp.maximum(z, z * jnp.float32(0.01))
                rows_v[e, pl.ds(j * LANES, LANES)] = r
            rows_v[e, pl.ds(D, LANES)] = cnt_vec
            return carry2

        lax.fori_loop(0, K, edge_body, 0)
        pltpu.sync_copy(rows_v, acc_sh.at[dst_v], add=True)
        return carry

    lax.fori_loop(0, CHUNKS, chunk_body, 0)
    plsc.subcore_barrier()
    pltpu.sync_copy(acc_sh.at[pl.ds(s * NODES_PT, NODES_PT)],
                    out_hbm.at[c, pl.ds(s * NODES_PT, NODES_PT)])


def _edge_sums(gb, gt_pad, src_p, dst_p, attr_flat, zeros):
    mesh = plsc.VectorSubcoreMesh(core_axis_name="c", subcore_axis_name="s",
                                  num_cores=NC, num_subcores=NS)
    f = pl.kernel(
        _edge_body,
        out_type=jax.ShapeDtypeStruct((NC, N_PAD, ROWW), jnp.float32),
        mesh=mesh,
        compiler_params=pltpu.CompilerParams(use_tc_tiling_on_sc=False,
                                             needs_layout_passes=False),
        scratch_types=[
            pltpu.VMEM((32 * D,), jnp.float32),
            pltpu.VMEM((N_PAD,), jnp.int32),
            pltpu.VMEM((K,), jnp.int32),
            pltpu.VMEM((K,), jnp.int32),
            pltpu.VMEM((3 * K,), jnp.float32),
            pltpu.VMEM((K, ROWW), jnp.float32),
            pltpu.VMEM_SHARED((N_PAD, ROWW), jnp.float32),
        ],
    )
    return f(gb, gt_pad, src_p, dst_p, attr_flat, zeros)


def _node_body(sc_ref, sa_ref, gt_ref, embc_ref, w2hc_ref, w2nc_ref, b2c_ref,
               wc1_ref, bc1_ref, wc2_ref, bc2_ref,
               emba_ref, w2ha_ref, w2na_ref, b2a_ref,
               wa1_ref, ba1_ref, wa2_ref, ba2_ref,
               logits_ref, value_ref):
    gt = gt_ref[...]
    iota = lax.broadcasted_iota(jnp.int32, (gt.shape[0], 32), 1)
    oh = (iota == gt).astype(jnp.float32)
    cnt = sc_ref[:, D:D + 1]
    inv = 1.0 / jnp.maximum(cnt, 1.0)

    h_c = jnp.dot(oh, embc_ref[...], preferred_element_type=jnp.float32)
    hn_c = sc_ref[:, :D] * inv
    hc = jnp.maximum(
        jnp.dot(h_c, w2hc_ref[...], preferred_element_type=jnp.float32)
        + jnp.dot(hn_c, w2nc_ref[...], preferred_element_type=jnp.float32)
        + b2c_ref[...], 0.0)
    vh = jnp.maximum(
        jnp.dot(hc, wc1_ref[...], preferred_element_type=jnp.float32)
        + bc1_ref[...], 0.0)
    value_ref[...] = (jnp.dot(vh, wc2_ref[...],
                              preferred_element_type=jnp.float32)
                      + bc2_ref[...])

    h_a = jnp.dot(oh, emba_ref[...], preferred_element_type=jnp.float32)
    hn_a = sa_ref[:, :D] * inv
    ha = jnp.maximum(
        jnp.dot(h_a, w2ha_ref[...], preferred_element_type=jnp.float32)
        + jnp.dot(hn_a, w2na_ref[...], preferred_element_type=jnp.float32)
        + b2a_ref[...], 0.0)
    lh = jnp.maximum(
        jnp.dot(ha, wa1_ref[...], preferred_element_type=jnp.float32)
        + ba1_ref[...], 0.0)
    logits_ref[...] = (jnp.dot(lh, wa2_ref[...],
                               preferred_element_type=jnp.float32)
                       + ba2_ref[...])


def _node_phase(sc, sa, gt2, embc, w2hc, w2nc, b2c, wc1, bc1, wc2p, bc2p,
                emba, w2ha, w2na, b2a, wa1, ba1, wa2, ba2):
    B = 512
    grid = (N_PAD // B,)
    full = lambda shape: pl.BlockSpec(shape, lambda i: (0, 0))
    return pl.pallas_call(
        _node_body,
        grid=grid,
        in_specs=[
            pl.BlockSpec((B, ROWW), lambda i: (i, 0)),
            pl.BlockSpec((B, ROWW), lambda i: (i, 0)),
            pl.BlockSpec((B, 1), lambda i: (i, 0)),
            full((32, D)), full((D, D)), full((D, D)), full((1, D)),
            full((D, D)), full((1, D)), full((D, 8)), full((1, 8)),
            full((32, D)), full((D, D)), full((D, D)), full((1, D)),
            full((D, D)), full((1, D)), full((D, NUM_OUT)),
            full((1, NUM_OUT)),
        ],
        out_specs=[
            pl.BlockSpec((B, NUM_OUT), lambda i: (i, 0)),
            pl.BlockSpec((B, 8), lambda i: (i, 0)),
        ],
        out_shape=[
            jax.ShapeDtypeStruct((N_PAD, NUM_OUT), jnp.float32),
            jax.ShapeDtypeStruct((N_PAD, 8), jnp.float32),
        ],
    )(sc, sa, gt2, embc, w2hc, w2nc, b2c, wc1, bc1, wc2p, bc2p,
      emba, w2ha, w2na, b2a, wa1, ba1, wa2, ba2)


def _emb_ext(emb):
    ext = jnp.zeros((32, 136), jnp.float32)
    ext = ext.at[:GATE_TYPES, :D].set(emb)
    ext = ext.at[GATE_TYPES:GATE_TYPES + 3, D:D + 3].set(jnp.eye(3))
    return ext


def kernel(gate_type, edge_index, edge_attr,
           emb_c, W1_c, W2_c, b2_c, Wc1, bc1, Wc2, bc2,
           emb_a, W1_a, W2_a, b2_a, Wa1, ba1, Wa2, ba2):
    embx_c = _emb_ext(emb_c)
    embx_a = _emb_ext(emb_a)
    w1f_c = jnp.zeros((136, D), jnp.float32).at[:D + 3].set(W1_c)
    w1f_a = jnp.zeros((136, D), jnp.float32).at[:D + 3].set(W1_a)

    gb = _prep_tables(embx_c, w1f_c, embx_a, w1f_a)
    gb = gb.reshape(NC, 32 * D)
    gb = gb.at[:, 29 * D].set(1.0)

    pad = E_PAD - E
    src_p = jnp.concatenate([edge_index[0],
                             jnp.zeros((pad,), jnp.int32)])
    dst_p = jnp.concatenate([edge_index[1],
                             N + (jnp.arange(pad, dtype=jnp.int32)
                                  % (N_PAD - N))])
    attr_p = jnp.concatenate([edge_attr,
                              jnp.zeros((pad, 3), jnp.float32)], axis=0)
    attr_flat = attr_p.reshape(E_PAD // K, K, 3).transpose(0, 2, 1).reshape(-1)
    gt_pad = jnp.concatenate(
        [gate_type, jnp.zeros((N_PAD - N,), jnp.int32)])
    zeros = jnp.zeros((NODES_PT, ROWW), jnp.float32)

    acc = _edge_sums(gb, gt_pad, src_p, dst_p, attr_flat, zeros)

    gt2 = gt_pad.reshape(N_PAD, 1)
    wc2p = jnp.zeros((D, 8), jnp.float32).at[:, :1].set(Wc2)
    bc2p = jnp.zeros((1, 8), jnp.float32).at[:, :1].set(bc2.reshape(1, 1))
    logits_p, value_p = _node_phase(
        acc[0], acc[1], gt2,
        embx_c[:, :D], W2_c[:D], W2_c[D:], b2_c.reshape(1, D),
        Wc1, bc1.reshape(1, D), wc2p, bc2p,
        embx_a[:, :D], W2_a[:D], W2_a[D:], b2_a.reshape(1, D),
        Wa1, ba1.reshape(1, D), Wa2, ba2.reshape(1, NUM_OUT))

    return (logits_p[:N], value_p[:N, :1])


# speedup vs baseline: 1.5907x; 1.1830x over previous
import functools

import jax
import jax.numpy as jnp
from jax import lax
from jax.experimental import pallas as pl
from jax.experimental.pallas import tpu as pltpu
from jax.experimental.pallas import tpu_sc as plsc

N = 10000
E = 320000
D = 128
GATE_TYPES = 26
NUM_OUT = 64

NC = 2
NS = 16
LANES = 16

N_PAD = 10240
ROWW = 144
K = 64
SLAB_E = 1024
SLAB_CH = SLAB_E // K
SLABS = 20
E_PT = SLABS * SLAB_E
E_PAD = NS * E_PT
NODES_PT = N_PAD // NS


def _prep_body(ec_ref, wc_ref, ea_ref, wa_ref, out_ref):
    out_ref[0] = jnp.dot(ec_ref[...], wc_ref[...],
                         preferred_element_type=jnp.float32)
    out_ref[1] = jnp.dot(ea_ref[...], wa_ref[...],
                         preferred_element_type=jnp.float32)


def _prep_tables(embx_c, w1f_c, embx_a, w1f_a):
    return pl.pallas_call(
        _prep_body,
        out_shape=jax.ShapeDtypeStruct((2, 32, D), jnp.float32),
    )(embx_c, w1f_c, embx_a, w1f_a)


def _edge_body(gb_hbm, gt_hbm, src_hbm, dst_hbm, attr_hbm, zeros_hbm,
               out_hbm, gb_v, gt_v, src_v, dst_v, a_v, rows0, rows1,
               acc_sh, sem0, sem1):
    c = lax.axis_index("c")
    s = lax.axis_index("s")

    pltpu.sync_copy(zeros_hbm, acc_sh.at[pl.ds(s * NODES_PT, NODES_PT)])
    pltpu.sync_copy(gb_hbm.at[c], gb_v)
    pltpu.sync_copy(gt_hbm, gt_v)
    plsc.subcore_barrier()

    iota = lax.iota(jnp.int32, LANES)
    b1 = [gb_v[pl.ds(26 * D + j * LANES, LANES)] for j in range(D // LANES)]
    b2 = [gb_v[pl.ds(27 * D + j * LANES, LANES)] for j in range(D // LANES)]
    b3 = [gb_v[pl.ds(28 * D + j * LANES, LANES)] for j in range(D // LANES)]
    cnt_vec = gb_v[pl.ds(29 * D, LANES)]

    rows = (rows0, rows1)
    sems = (sem0, sem1)

    def compute_chunk(chunk, buf):
        ebase = chunk * K

        def edge_body(e, carry2):
            eg = ebase + e
            e16 = jnp.full((LANES,), eg, jnp.int32)
            s_b = plsc.load_gather(src_v, [e16])
            t_b = plsc.load_gather(gt_v, [s_b])
            a1 = plsc.load_gather(a_v, [e16])
            a2 = plsc.load_gather(a_v, [e16 + SLAB_E])
            a3 = plsc.load_gather(a_v, [e16 + 2 * SLAB_E])
            gbase = t_b * D + iota
            for j in range(D // LANES):
                g = plsc.load_gather(gb_v, [gbase + j * LANES])
                z = g + a1 * b1[j] + a2 * b2[j] + a3 * b3[j]
                rr = jnp.maximum(z, z * jnp.float32(0.01))
                buf[e, pl.ds(j * LANES, LANES)] = rr
            buf[e, pl.ds(D, LANES)] = cnt_vec
            return carry2

        lax.fori_loop(0, K, edge_body, 0, unroll=2)

    def slab_body(sl, hcarry):
        @pl.when(sl > 0)
        def _drain():
            for b in range(2):
                pltpu.make_async_copy(
                    rows[b], acc_sh.at[dst_v.at[SLAB_CH - 2 + b]],
                    sems[b]).wait()

        sbase = s * E_PT + sl * SLAB_E
        pltpu.sync_copy(src_hbm.at[pl.ds(sbase, SLAB_E)], src_v)
        pltpu.sync_copy(dst_hbm.at[pl.ds(sbase // K, SLAB_CH)], dst_v)
        pltpu.sync_copy(attr_hbm.at[pl.ds(3 * sbase, 3 * SLAB_E)], a_v)

        def outer_body(g, carry):
            for b in range(2):
                chunk = g * 2 + b

                @pl.when(g > 0)
                def _wait():
                    pltpu.make_async_copy(
                        rows[b], acc_sh.at[dst_v.at[chunk]], sems[b]).wait()

                compute_chunk(chunk, rows[b])
                pltpu.async_copy(rows[b], acc_sh.at[dst_v.at[chunk]],
                                 sems[b], add=True)
            return carry

        lax.fori_loop(0, SLAB_CH // 2, outer_body, 0)
        return hcarry

    lax.fori_loop(0, SLABS, slab_body, 0)
    for b in range(2):
        pltpu.make_async_copy(
            rows[b], acc_sh.at[dst_v.at[SLAB_CH - 2 + b]], sems[b]).wait()

    plsc.subcore_barrier()
    pltpu.sync_copy(acc_sh.at[pl.ds(s * NODES_PT, NODES_PT)],
                    out_hbm.at[c, pl.ds(s * NODES_PT, NODES_PT)])


def _edge_sums(gb, gt_pad, src_h, dst_h, attr_h, zeros):
    mesh = plsc.VectorSubcoreMesh(core_axis_name="c", subcore_axis_name="s",
                                  num_cores=NC, num_subcores=NS)
    f = pl.kernel(
        _edge_body,
        out_type=jax.ShapeDtypeStruct((NC, N_PAD, ROWW), jnp.float32),
        mesh=mesh,
        compiler_params=pltpu.CompilerParams(use_tc_tiling_on_sc=False,
                                             needs_layout_passes=False),
        scratch_types=[
            pltpu.VMEM((32 * D,), jnp.float32),
            pltpu.VMEM((N_PAD,), jnp.int32),
            pltpu.VMEM((SLAB_E,), jnp.int32),
            pltpu.VMEM((SLAB_CH, K), jnp.int32),
            pltpu.VMEM((3 * SLAB_E,), jnp.float32),
            pltpu.VMEM((K, ROWW), jnp.float32),
            pltpu.VMEM((K, ROWW), jnp.float32),
            pltpu.VMEM_SHARED((N_PAD, ROWW), jnp.float32),
            pltpu.SemaphoreType.DMA,
            pltpu.SemaphoreType.DMA,
        ],
    )
    return f(gb, gt_pad, src_h, dst_h, attr_h, zeros)


def _node_body(sc_ref, sa_ref, gt_ref, embc_ref, w2hc_ref, w2nc_ref, b2c_ref,
               wc1_ref, bc1_ref, wc2_ref, bc2_ref,
               emba_ref, w2ha_ref, w2na_ref, b2a_ref,
               wa1_ref, ba1_ref, wa2_ref, ba2_ref,
               logits_ref, value_ref):
    gt = gt_ref[...]
    iota = lax.broadcasted_iota(jnp.int32, (gt.shape[0], 32), 1)
    oh = (iota == gt).astype(jnp.float32)
    cnt = sc_ref[:, D:D + 1]
    inv = 1.0 / jnp.maximum(cnt, 1.0)

    h_c = jnp.dot(oh, embc_ref[...], preferred_element_type=jnp.float32)
    hn_c = sc_ref[:, :D] * inv
    hc = jnp.maximum(
        jnp.dot(h_c, w2hc_ref[...], preferred_element_type=jnp.float32)
        + jnp.dot(hn_c, w2nc_ref[...], preferred_element_type=jnp.float32)
        + b2c_ref[...], 0.0)
    vh = jnp.maximum(
        jnp.dot(hc, wc1_ref[...], preferred_element_type=jnp.float32)
        + bc1_ref[...], 0.0)
    value_ref[...] = (jnp.dot(vh, wc2_ref[...],
                              preferred_element_type=jnp.float32)
                      + bc2_ref[...])

    h_a = jnp.dot(oh, emba_ref[...], preferred_element_type=jnp.float32)
    hn_a = sa_ref[:, :D] * inv
    ha = jnp.maximum(
        jnp.dot(h_a, w2ha_ref[...], preferred_element_type=jnp.float32)
        + jnp.dot(hn_a, w2na_ref[...], preferred_element_type=jnp.float32)
        + b2a_ref[...], 0.0)
    lh = jnp.maximum(
        jnp.dot(ha, wa1_ref[...], preferred_element_type=jnp.float32)
        + ba1_ref[...], 0.0)
    logits_ref[...] = (jnp.dot(lh, wa2_ref[...],
                               preferred_element_type=jnp.float32)
                       + ba2_ref[...])


def _node_phase(sc, sa, gt2, embc, w2hc, w2nc, b2c, wc1, bc1, wc2p, bc2p,
                emba, w2ha, w2na, b2a, wa1, ba1, wa2, ba2):
    B = 512
    grid = (N_PAD // B,)
    full = lambda shape: pl.BlockSpec(shape, lambda i: (0, 0))
    return pl.pallas_call(
        _node_body,
        grid=grid,
        in_specs=[
            pl.BlockSpec((B, ROWW), lambda i: (i, 0)),
            pl.BlockSpec((B, ROWW), lambda i: (i, 0)),
            pl.BlockSpec((B, 1), lambda i: (i, 0)),
            full((32, D)), full((D, D)), full((D, D)), full((1, D)),
            full((D, D)), full((1, D)), full((D, 8)), full((1, 8)),
            full((32, D)), full((D, D)), full((D, D)), full((1, D)),
            full((D, D)), full((1, D)), full((D, NUM_OUT)),
            full((1, NUM_OUT)),
        ],
        out_specs=[
            pl.BlockSpec((B, NUM_OUT), lambda i: (i, 0)),
            pl.BlockSpec((B, 8), lambda i: (i, 0)),
        ],
        out_shape=[
            jax.ShapeDtypeStruct((N_PAD, NUM_OUT), jnp.float32),
            jax.ShapeDtypeStruct((N_PAD, 8), jnp.float32),
        ],
    )(sc, sa, gt2, embc, w2hc, w2nc, b2c, wc1, bc1, wc2p, bc2p,
      emba, w2ha, w2na, b2a, wa1, ba1, wa2, ba2)


def _emb_ext(emb):
    ext = jnp.zeros((32, 136), jnp.float32)
    ext = ext.at[:GATE_TYPES, :D].set(emb)
    ext = ext.at[GATE_TYPES:GATE_TYPES + 3, D:D + 3].set(jnp.eye(3))
    return ext


def kernel(gate_type, edge_index, edge_attr,
           emb_c, W1_c, W2_c, b2_c, Wc1, bc1, Wc2, bc2,
           emb_a, W1_a, W2_a, b2_a, Wa1, ba1, Wa2, ba2):
    embx_c = _emb_ext(emb_c)
    embx_a = _emb_ext(emb_a)
    w1f_c = jnp.zeros((136, D), jnp.float32).at[:D + 3].set(W1_c)
    w1f_a = jnp.zeros((136, D), jnp.float32).at[:D + 3].set(W1_a)

    gb = _prep_tables(embx_c, w1f_c, embx_a, w1f_a)
    gb = gb.reshape(NC, 32 * D)
    gb = gb.at[:, 29 * D].set(1.0)

    pad = E_PAD - E
    src_p = jnp.concatenate([edge_index[0],
                             jnp.zeros((pad,), jnp.int32)])
    dst_p = jnp.concatenate([edge_index[1],
                             N + (jnp.arange(pad, dtype=jnp.int32)
                                  % (N_PAD - N))])
    attr_p = jnp.concatenate([edge_attr,
                              jnp.zeros((pad, 3), jnp.float32)], axis=0)
    src_h = src_p
    dst_h = dst_p.reshape(E_PAD // K, K)
    attr_h = attr_p.reshape(E_PAD // SLAB_E, SLAB_E, 3) \
                   .transpose(0, 2, 1).reshape(-1)
    gt_pad = jnp.concatenate(
        [gate_type, jnp.zeros((N_PAD - N,), jnp.int32)])
    zeros = jnp.zeros((NODES_PT, ROWW), jnp.float32)

    acc = _edge_sums(gb, gt_pad, src_h, dst_h, attr_h, zeros)

    gt2 = gt_pad.reshape(N_PAD, 1)
    wc2p = jnp.zeros((D, 8), jnp.float32).at[:, :1].set(Wc2)
    bc2p = jnp.zeros((1, 8), jnp.float32).at[:, :1].set(bc2.reshape(1, 1))
    logits_p, value_p = _node_phase(
        acc[0], acc[1], gt2,
        embx_c[:, :D], W2_c[:D], W2_c[D:], b2_c.reshape(1, D),
        Wc1, bc1.reshape(1, D), wc2p, bc2p,
        embx_a[:, :D], W2_a[:D], W2_a[D:], b2_a.reshape(1, D),
        Wa1, ba1.reshape(1, D), Wa2, ba2.reshape(1, NUM_OUT))

    return (logits_p[:N], value_p[:N, :1])


# speedup vs baseline: 5.0767x; 3.1914x over previous
import functools

import jax
import jax.numpy as jnp
from jax import lax
from jax.experimental import pallas as pl
from jax.experimental.pallas import tpu as pltpu
from jax.experimental.pallas import tpu_sc as plsc

N = 10000
E = 320000
D = 128
GATE_TYPES = 26
NUM_OUT = 64

NC = 2
NS = 16
LANES = 16

N_PAD = 10240
ROWW = 144
K = 64
SLAB_E = 1024
SLAB_CH = SLAB_E // K
SLABS = 20
E_PT = SLABS * SLAB_E
E_PAD = NS * E_PT
NODES_PT = N_PAD // NS


def _prep_body(ec_ref, wc_ref, ea_ref, wa_ref, out_ref):
    out_ref[0] = jnp.dot(ec_ref[...], wc_ref[...],
                         preferred_element_type=jnp.float32)
    out_ref[1] = jnp.dot(ea_ref[...], wa_ref[...],
                         preferred_element_type=jnp.float32)


def _prep_tables(embx_c, w1f_c, embx_a, w1f_a):
    return pl.pallas_call(
        _prep_body,
        out_shape=jax.ShapeDtypeStruct((2, 32, D), jnp.float32),
    )(embx_c, w1f_c, embx_a, w1f_a)


def _edge_body(gb_hbm, gt_hbm, src_hbm, dst_hbm, attr_hbm, zeros_hbm,
               out_hbm, gb_v, gt_v, src_v, dst_v, a_v, rows0, rows1,
               acc_sh, sem0, sem1):
    c = lax.axis_index("c")
    s = lax.axis_index("s")

    pltpu.sync_copy(zeros_hbm, acc_sh.at[pl.ds(s * NODES_PT, NODES_PT)])
    pltpu.sync_copy(gb_hbm.at[c], gb_v)
    pltpu.sync_copy(gt_hbm, gt_v)
    plsc.subcore_barrier()

    iota = lax.iota(jnp.int32, LANES)
    b1 = [gb_v[pl.ds(26 * D + j * LANES, LANES)] for j in range(D // LANES)]
    b2 = [gb_v[pl.ds(27 * D + j * LANES, LANES)] for j in range(D // LANES)]
    b3 = [gb_v[pl.ds(28 * D + j * LANES, LANES)] for j in range(D // LANES)]
    cnt_vec = gb_v[pl.ds(29 * D, LANES)]

    rows = (rows0, rows1)
    sems = (sem0, sem1)

    def compute_chunk(chunk, buf):
        ebase = chunk * K

        @plsc.parallel_loop(0, K)
        def edge_body(e):
            eg = ebase + e
            e16 = jnp.full((LANES,), eg, jnp.int32)
            s_b = plsc.load_gather(src_v, [e16])
            t_b = plsc.load_gather(gt_v, [s_b])
            a1 = plsc.load_gather(a_v, [e16])
            a2 = plsc.load_gather(a_v, [e16 + SLAB_E])
            a3 = plsc.load_gather(a_v, [e16 + 2 * SLAB_E])
            gbase = t_b * D + iota
            for j in range(D // LANES):
                g = plsc.load_gather(gb_v, [gbase + j * LANES])
                z = g + a1 * b1[j] + a2 * b2[j] + a3 * b3[j]
                rr = jnp.maximum(z, z * jnp.float32(0.01))
                buf[e, pl.ds(j * LANES, LANES)] = rr
            buf[e, pl.ds(D, LANES)] = cnt_vec

    def slab_body(sl, hcarry):
        @pl.when(sl > 0)
        def _drain():
            for b in range(2):
                pltpu.make_async_copy(
                    rows[b], acc_sh.at[dst_v.at[SLAB_CH - 2 + b]],
                    sems[b]).wait()

        sbase = s * E_PT + sl * SLAB_E
        pltpu.sync_copy(src_hbm.at[pl.ds(sbase, SLAB_E)], src_v)
        pltpu.sync_copy(dst_hbm.at[pl.ds(sbase // K, SLAB_CH)], dst_v)
        pltpu.sync_copy(attr_hbm.at[pl.ds(3 * sbase, 3 * SLAB_E)], a_v)

        def outer_body(g, carry):
            for b in range(2):
                chunk = g * 2 + b

                @pl.when(g > 0)
                def _wait():
                    pltpu.make_async_copy(
                        rows[b], acc_sh.at[dst_v.at[chunk]], sems[b]).wait()

                compute_chunk(chunk, rows[b])
                pltpu.async_copy(rows[b], acc_sh.at[dst_v.at[chunk]],
                                 sems[b], add=True)
            return carry

        lax.fori_loop(0, SLAB_CH // 2, outer_body, 0)
        return hcarry

    lax.fori_loop(0, SLABS, slab_body, 0)
    for b in range(2):
        pltpu.make_async_copy(
            rows[b], acc_sh.at[dst_v.at[SLAB_CH - 2 + b]], sems[b]).wait()

    plsc.subcore_barrier()
    pltpu.sync_copy(acc_sh.at[pl.ds(s * NODES_PT, NODES_PT)],
                    out_hbm.at[c, pl.ds(s * NODES_PT, NODES_PT)])


def _edge_sums(gb, gt_pad, src_h, dst_h, attr_h, zeros):
    mesh = plsc.VectorSubcoreMesh(core_axis_name="c", subcore_axis_name="s",
                                  num_cores=NC, num_subcores=NS)
    f = pl.kernel(
        _edge_body,
        out_type=jax.ShapeDtypeStruct((NC, N_PAD, ROWW), jnp.float32),
        mesh=mesh,
        compiler_params=pltpu.CompilerParams(use_tc_tiling_on_sc=False,
                                             needs_layout_passes=False),
        scratch_types=[
            pltpu.VMEM((32 * D,), jnp.float32),
            pltpu.VMEM((N_PAD,), jnp.int32),
            pltpu.VMEM((SLAB_E,), jnp.int32),
            pltpu.VMEM((SLAB_CH, K), jnp.int32),
            pltpu.VMEM((3 * SLAB_E,), jnp.float32),
            pltpu.VMEM((K, ROWW), jnp.float32),
            pltpu.VMEM((K, ROWW), jnp.float32),
            pltpu.VMEM_SHARED((N_PAD, ROWW), jnp.float32),
            pltpu.SemaphoreType.DMA,
            pltpu.SemaphoreType.DMA,
        ],
    )
    return f(gb, gt_pad, src_h, dst_h, attr_h, zeros)


def _node_body(sc_ref, sa_ref, gt_ref, embc_ref, w2hc_ref, w2nc_ref, b2c_ref,
               wc1_ref, bc1_ref, wc2_ref, bc2_ref,
               emba_ref, w2ha_ref, w2na_ref, b2a_ref,
               wa1_ref, ba1_ref, wa2_ref, ba2_ref,
               logits_ref, value_ref):
    gt = gt_ref[...]
    iota = lax.broadcasted_iota(jnp.int32, (gt.shape[0], 32), 1)
    oh = (iota == gt).astype(jnp.float32)
    cnt = sc_ref[:, D:D + 1]
    inv = 1.0 / jnp.maximum(cnt, 1.0)

    h_c = jnp.dot(oh, embc_ref[...], preferred_element_type=jnp.float32)
    hn_c = sc_ref[:, :D] * inv
    hc = jnp.maximum(
        jnp.dot(h_c, w2hc_ref[...], preferred_element_type=jnp.float32)
        + jnp.dot(hn_c, w2nc_ref[...], preferred_element_type=jnp.float32)
        + b2c_ref[...], 0.0)
    vh = jnp.maximum(
        jnp.dot(hc, wc1_ref[...], preferred_element_type=jnp.float32)
        + bc1_ref[...], 0.0)
    value_ref[...] = (jnp.dot(vh, wc2_ref[...],
                              preferred_element_type=jnp.float32)
                      + bc2_ref[...])

    h_a = jnp.dot(oh, emba_ref[...], preferred_element_type=jnp.float32)
    hn_a = sa_ref[:, :D] * inv
    ha = jnp.maximum(
        jnp.dot(h_a, w2ha_ref[...], preferred_element_type=jnp.float32)
        + jnp.dot(hn_a, w2na_ref[...], preferred_element_type=jnp.float32)
        + b2a_ref[...], 0.0)
    lh = jnp.maximum(
        jnp.dot(ha, wa1_ref[...], preferred_element_type=jnp.float32)
        + ba1_ref[...], 0.0)
    logits_ref[...] = (jnp.dot(lh, wa2_ref[...],
                               preferred_element_type=jnp.float32)
                       + ba2_ref[...])


def _node_phase(sc, sa, gt2, embc, w2hc, w2nc, b2c, wc1, bc1, wc2p, bc2p,
                emba, w2ha, w2na, b2a, wa1, ba1, wa2, ba2):
    B = 512
    grid = (N_PAD // B,)
    full = lambda shape: pl.BlockSpec(shape, lambda i: (0, 0))
    return pl.pallas_call(
        _node_body,
        grid=grid,
        in_specs=[
            pl.BlockSpec((B, ROWW), lambda i: (i, 0)),
            pl.BlockSpec((B, ROWW), lambda i: (i, 0)),
            pl.BlockSpec((B, 1), lambda i: (i, 0)),
            full((32, D)), full((D, D)), full((D, D)), full((1, D)),
            full((D, D)), full((1, D)), full((D, 8)), full((1, 8)),
            full((32, D)), full((D, D)), full((D, D)), full((1, D)),
            full((D, D)), full((1, D)), full((D, NUM_OUT)),
            full((1, NUM_OUT)),
        ],
        out_specs=[
            pl.BlockSpec((B, NUM_OUT), lambda i: (i, 0)),
            pl.BlockSpec((B, 8), lambda i: (i, 0)),
        ],
        out_shape=[
            jax.ShapeDtypeStruct((N_PAD, NUM_OUT), jnp.float32),
            jax.ShapeDtypeStruct((N_PAD, 8), jnp.float32),
        ],
    )(sc, sa, gt2, embc, w2hc, w2nc, b2c, wc1, bc1, wc2p, bc2p,
      emba, w2ha, w2na, b2a, wa1, ba1, wa2, ba2)


def _emb_ext(emb):
    ext = jnp.zeros((32, 136), jnp.float32)
    ext = ext.at[:GATE_TYPES, :D].set(emb)
    ext = ext.at[GATE_TYPES:GATE_TYPES + 3, D:D + 3].set(jnp.eye(3))
    return ext


def kernel(gate_type, edge_index, edge_attr,
           emb_c, W1_c, W2_c, b2_c, Wc1, bc1, Wc2, bc2,
           emb_a, W1_a, W2_a, b2_a, Wa1, ba1, Wa2, ba2):
    embx_c = _emb_ext(emb_c)
    embx_a = _emb_ext(emb_a)
    w1f_c = jnp.zeros((136, D), jnp.float32).at[:D + 3].set(W1_c)
    w1f_a = jnp.zeros((136, D), jnp.float32).at[:D + 3].set(W1_a)

    gb = _prep_tables(embx_c, w1f_c, embx_a, w1f_a)
    gb = gb.reshape(NC, 32 * D)
    gb = gb.at[:, 29 * D].set(1.0)

    pad = E_PAD - E
    src_p = jnp.concatenate([edge_index[0],
                             jnp.zeros((pad,), jnp.int32)])
    dst_p = jnp.concatenate([edge_index[1],
                             N + (jnp.arange(pad, dtype=jnp.int32)
                                  % (N_PAD - N))])
    attr_p = jnp.concatenate([edge_attr,
                              jnp.zeros((pad, 3), jnp.float32)], axis=0)
    src_h = src_p
    dst_h = dst_p.reshape(E_PAD // K, K)
    attr_h = attr_p.reshape(E_PAD // SLAB_E, SLAB_E, 3) \
                   .transpose(0, 2, 1).reshape(-1)
    gt_pad = jnp.concatenate(
        [gate_type, jnp.zeros((N_PAD - N,), jnp.int32)])
    zeros = jnp.zeros((NODES_PT, ROWW), jnp.float32)

    acc = _edge_sums(gb, gt_pad, src_h, dst_h, attr_h, zeros)

    gt2 = gt_pad.reshape(N_PAD, 1)
    wc2p = jnp.zeros((D, 8), jnp.float32).at[:, :1].set(Wc2)
    bc2p = jnp.zeros((1, 8), jnp.float32).at[:, :1].set(bc2.reshape(1, 1))
    logits_p, value_p = _node_phase(
        acc[0], acc[1], gt2,
        embx_c[:, :D], W2_c[:D], W2_c[D:], b2_c.reshape(1, D),
        Wc1, bc1.reshape(1, D), wc2p, bc2p,
        embx_a[:, :D], W2_a[:D], W2_a[D:], b2_a.reshape(1, D),
        Wa1, ba1.reshape(1, D), Wa2, ba2.reshape(1, NUM_OUT))

    return (logits_p[:N], value_p[:N, :1])


# speedup vs baseline: 7.6709x; 1.5110x over previous
import jax
import jax.numpy as jnp
from jax import lax
from jax.experimental import pallas as pl
from jax.experimental.pallas import tpu as pltpu
from jax.experimental.pallas import tpu_sc as plsc

N = 10000
E = 320000
D = 128
GATE_TYPES = 26
NUM_OUT = 64

NC = 2
NS = 16
LANES = 16

N_PAD = 10240
ROWW = 144
K = 64
SLAB_E = 1024
SLAB_CH = SLAB_E // K
SLABS = 20
E_PT = SLABS * SLAB_E
E_PAD = NS * E_PT
NODES_PT = N_PAD // NS


def _prep_body(ec_ref, wc_ref, ea_ref, wa_ref, out_ref):
    out_ref[0] = jnp.dot(ec_ref[...], wc_ref[...],
                         preferred_element_type=jnp.float32)
    out_ref[1] = jnp.dot(ea_ref[...], wa_ref[...],
                         preferred_element_type=jnp.float32)


def _prep_tables(embx_c, w1f_c, embx_a, w1f_a):
    return pl.pallas_call(
        _prep_body,
        out_shape=jax.ShapeDtypeStruct((2, 32, D), jnp.float32),
    )(embx_c, w1f_c, embx_a, w1f_a)


GW = 64


def _edge_body(gb_hbm, gt_hbm, src_hbm, dst_hbm, attr_hbm, cnt_hbm,
               zeros_hbm, out_hbm, gb_v, gt_v, src_v, t_v, dst_v, a_v,
               cnt_v, rows0, rows1, acc_sh, sem0, sem1):
    c = lax.axis_index("c")
    s = lax.axis_index("s")

    pltpu.sync_copy(zeros_hbm, acc_sh.at[pl.ds(s * NODES_PT, NODES_PT)])
    pltpu.sync_copy(gb_hbm.at[c], gb_v)
    pltpu.sync_copy(gt_hbm, gt_v)
    pltpu.sync_copy(cnt_hbm, cnt_v)
    plsc.subcore_barrier()

    iota = lax.iota(jnp.int32, LANES)
    NG = D // (2 * LANES)
    bb = [[plsc.bitcast(gb_v[pl.ds((26 + i) * GW + j * LANES, LANES)],
                        jnp.bfloat16)
           for j in range(NG)] for i in range(3)]
    cnt_vec = cnt_v[...]

    rows = (rows0, rows1)
    sems = (sem0, sem1)

    for buf in rows:
        @plsc.parallel_loop(0, K)
        def _init_cnt(rr):
            buf[rr, pl.ds(D, LANES)] = cnt_vec

    def compute_chunk(chunk, buf):
        ebase = chunk * K

        @plsc.parallel_loop(0, K)
        def edge_body(e):
            eg = ebase + e
            e16 = jnp.full((LANES,), eg, jnp.int32)
            t_b = plsc.load_gather(t_v, [e16])
            a1 = plsc.bitcast(plsc.load_gather(a_v, [e16]), jnp.bfloat16)
            a2 = plsc.bitcast(plsc.load_gather(a_v, [e16 + SLAB_E]),
                              jnp.bfloat16)
            a3 = plsc.bitcast(plsc.load_gather(a_v, [e16 + 2 * SLAB_E]),
                              jnp.bfloat16)
            gbase = t_b * GW + iota
            for j in range(NG):
                gw = plsc.load_gather(gb_v, [gbase + j * LANES])
                g = plsc.bitcast(gw, jnp.bfloat16)
                z = g + a1 * bb[0][j] + a2 * bb[1][j] + a3 * bb[2][j]
                rr = jnp.maximum(z, z * jnp.bfloat16(0.01))
                lo, hi = plsc.unpack(rr, format=plsc.PackFormat.INTERLEAVED)
                buf[e, pl.ds(2 * j * LANES, LANES)] = lo
                buf[e, pl.ds((2 * j + 1) * LANES, LANES)] = hi

    def slab_body(sl, hcarry):
        @pl.when(sl > 0)
        def _drain():
            for b in range(2):
                pltpu.make_async_copy(
                    rows[b], acc_sh.at[dst_v.at[SLAB_CH - 2 + b]],
                    sems[b]).wait()

        sbase = s * E_PT + sl * SLAB_E
        pltpu.sync_copy(src_hbm.at[pl.ds(sbase, SLAB_E)], src_v)
        pltpu.sync_copy(dst_hbm.at[pl.ds(sbase // K, SLAB_CH)], dst_v)
        pltpu.sync_copy(attr_hbm.at[pl.ds(3 * sbase, 3 * SLAB_E)], a_v)

        @plsc.parallel_loop(0, SLAB_E // LANES)
        def _pre(g):
            s16 = src_v[pl.ds(g * LANES, LANES)]
            t_v[pl.ds(g * LANES, LANES)] = plsc.load_gather(gt_v, [s16])

        def outer_body(g, carry):
            for b in range(2):
                chunk = g * 2 + b

                @pl.when(g > 0)
                def _wait():
                    pltpu.make_async_copy(
                        rows[b], acc_sh.at[dst_v.at[chunk]], sems[b]).wait()

                compute_chunk(chunk, rows[b])
                pltpu.async_copy(rows[b], acc_sh.at[dst_v.at[chunk]],
                                 sems[b], add=True)
            return carry

        lax.fori_loop(0, SLAB_CH // 2, outer_body, 0)
        return hcarry

    lax.fori_loop(0, SLABS, slab_body, 0)
    for b in range(2):
        pltpu.make_async_copy(
            rows[b], acc_sh.at[dst_v.at[SLAB_CH - 2 + b]], sems[b]).wait()

    plsc.subcore_barrier()
    pltpu.sync_copy(acc_sh.at[pl.ds(s * NODES_PT, NODES_PT)],
                    out_hbm.at[c, pl.ds(s * NODES_PT, NODES_PT)])


def _edge_sums(gbw, gt_pad, src_h, dst_h, attr_h, cnt16, zeros):
    mesh = plsc.VectorSubcoreMesh(core_axis_name="c", subcore_axis_name="s",
                                  num_cores=NC, num_subcores=NS)
    f = pl.kernel(
        _edge_body,
        out_type=jax.ShapeDtypeStruct((NC, N_PAD, ROWW), jnp.float32),
        mesh=mesh,
        compiler_params=pltpu.CompilerParams(use_tc_tiling_on_sc=False,
                                             needs_layout_passes=False),
        scratch_types=[
            pltpu.VMEM((32 * GW,), jnp.int32),
            pltpu.VMEM((N_PAD,), jnp.int32),
            pltpu.VMEM((SLAB_E,), jnp.int32),
            pltpu.VMEM((SLAB_E,), jnp.int32),
            pltpu.VMEM((SLAB_CH, K), jnp.int32),
            pltpu.VMEM((3 * SLAB_E,), jnp.int32),
            pltpu.VMEM((LANES,), jnp.float32),
            pltpu.VMEM((K, ROWW), jnp.float32),
            pltpu.VMEM((K, ROWW), jnp.float32),
            pltpu.VMEM_SHARED((N_PAD, ROWW), jnp.float32),
            pltpu.SemaphoreType.DMA,
            pltpu.SemaphoreType.DMA,
        ],
    )
    return f(gbw, gt_pad, src_h, dst_h, attr_h, cnt16, zeros)


def _node_body(sc_ref, sa_ref, gt_ref, embc_ref, w2hc_ref, w2nc_ref, b2c_ref,
               wc1_ref, bc1_ref, wc2_ref, bc2_ref,
               emba_ref, w2ha_ref, w2na_ref, b2a_ref,
               wa1_ref, ba1_ref, wa2_ref, ba2_ref,
               logits_ref, value_ref):
    gt = gt_ref[...]
    iota = lax.broadcasted_iota(jnp.int32, (gt.shape[0], 32), 1)
    oh = (iota == gt).astype(jnp.float32)
    cnt = sc_ref[0, :, D:D + 1]
    inv = 1.0 / jnp.maximum(cnt, 1.0)

    h_c = jnp.dot(oh, embc_ref[...], preferred_element_type=jnp.float32)
    hn_c = sc_ref[0, :, :D] * inv
    hc = jnp.maximum(
        jnp.dot(h_c, w2hc_ref[...], preferred_element_type=jnp.float32)
        + jnp.dot(hn_c, w2nc_ref[...], preferred_element_type=jnp.float32)
        + b2c_ref[...], 0.0)
    vh = jnp.maximum(
        jnp.dot(hc, wc1_ref[...], preferred_element_type=jnp.float32)
        + bc1_ref[...], 0.0)
    value_ref[...] = (jnp.dot(vh, wc2_ref[...],
                              preferred_element_type=jnp.float32)
                      + bc2_ref[...])

    h_a = jnp.dot(oh, emba_ref[...], preferred_element_type=jnp.float32)
    hn_a = sa_ref[0, :, :D] * inv
    ha = jnp.maximum(
        jnp.dot(h_a, w2ha_ref[...], preferred_element_type=jnp.float32)
        + jnp.dot(hn_a, w2na_ref[...], preferred_element_type=jnp.float32)
        + b2a_ref[...], 0.0)
    lh = jnp.maximum(
        jnp.dot(ha, wa1_ref[...], preferred_element_type=jnp.float32)
        + ba1_ref[...], 0.0)
    logits_ref[...] = (jnp.dot(lh, wa2_ref[...],
                               preferred_element_type=jnp.float32)
                       + ba2_ref[...])


def _node_phase(sc, sa, gt2, embc, w2hc, w2nc, b2c, wc1, bc1, wc2p, bc2p,
                emba, w2ha, w2na, b2a, wa1, ba1, wa2, ba2):
    B = 512
    grid = (N_PAD // B,)
    full = lambda shape: pl.BlockSpec(shape, lambda i: (0, 0))
    return pl.pallas_call(
        _node_body,
        grid=grid,
        in_specs=[
            pl.BlockSpec((1, B, ROWW), lambda i: (0, i, 0)),
            pl.BlockSpec((1, B, ROWW), lambda i: (1, i, 0)),
            pl.BlockSpec((B, 1), lambda i: (i, 0)),
            full((32, D)), full((D, D)), full((D, D)), full((1, D)),
            full((D, D)), full((1, D)), full((D, 8)), full((1, 8)),
            full((32, D)), full((D, D)), full((D, D)), full((1, D)),
            full((D, D)), full((1, D)), full((D, NUM_OUT)),
            full((1, NUM_OUT)),
        ],
        out_specs=[
            pl.BlockSpec((B, NUM_OUT), lambda i: (i, 0)),
            pl.BlockSpec((B, 8), lambda i: (i, 0)),
        ],
        out_shape=[
            jax.ShapeDtypeStruct((N, NUM_OUT), jnp.float32),
            jax.ShapeDtypeStruct((N, 8), jnp.float32),
        ],
    )(sc, sa, gt2, embc, w2hc, w2nc, b2c, wc1, bc1, wc2p, bc2p,
      emba, w2ha, w2na, b2a, wa1, ba1, wa2, ba2)


def _emb_ext(emb):
    ext = jnp.zeros((32, 136), jnp.float32)
    ext = ext.at[:GATE_TYPES, :D].set(emb)
    ext = ext.at[GATE_TYPES:GATE_TYPES + 3, D:D + 3].set(jnp.eye(3))
    return ext


def kernel(gate_type, edge_index, edge_attr,
           emb_c, W1_c, W2_c, b2_c, Wc1, bc1, Wc2, bc2,
           emb_a, W1_a, W2_a, b2_a, Wa1, ba1, Wa2, ba2):
    embx_c = _emb_ext(emb_c)
    embx_a = _emb_ext(emb_a)
    w1f_c = jnp.zeros((136, D), jnp.float32).at[:D + 3].set(W1_c)
    w1f_a = jnp.zeros((136, D), jnp.float32).at[:D + 3].set(W1_a)

    gb = _prep_tables(embx_c, w1f_c, embx_a, w1f_a)
    gbits = jax.lax.bitcast_convert_type(
        gb.astype(jnp.bfloat16), jnp.uint16).astype(jnp.uint32)
    gr = gbits.reshape(NC, 32, D // 32, 2, 16)
    gbw = jax.lax.bitcast_convert_type(
        gr[:, :, :, 0, :] | (gr[:, :, :, 1, :] << 16), jnp.int32)
    gbw = gbw.reshape(NC, 32 * GW)
    cnt16 = jnp.zeros((LANES,), jnp.float32).at[0].set(1.0)

    pad = E_PAD - E
    src_p = jnp.concatenate([edge_index[0],
                             jnp.zeros((pad,), jnp.int32)])
    dst_p = jnp.concatenate([edge_index[1],
                             N + (jnp.arange(pad, dtype=jnp.int32)
                                  % (N_PAD - N))])
    attr_p = jnp.concatenate([edge_attr,
                              jnp.zeros((pad, 3), jnp.float32)], axis=0)
    src_h = src_p
    dst_h = dst_p.reshape(E_PAD // K, K)
    abits = jax.lax.bitcast_convert_type(
        attr_p.astype(jnp.bfloat16), jnp.uint16).astype(jnp.uint32)
    aw = jax.lax.bitcast_convert_type(abits | (abits << 16), jnp.int32)
    attr_h = aw.reshape(E_PAD // SLAB_E, SLAB_E, 3) \
               .transpose(0, 2, 1).reshape(-1)
    gt_pad = jnp.concatenate(
        [gate_type, jnp.zeros((N_PAD - N,), jnp.int32)])
    zeros = jnp.zeros((NODES_PT, ROWW), jnp.float32)

    acc = _edge_sums(gbw, gt_pad, src_h, dst_h, attr_h, cnt16, zeros)

    gt2 = gt_pad.reshape(N_PAD, 1)
    wc2p = jnp.zeros((D, 8), jnp.float32).at[:, :1].set(Wc2)
    bc2p = jnp.zeros((1, 8), jnp.float32).at[:, :1].set(bc2.reshape(1, 1))
    logits_p, value_p = _node_phase(
        acc, acc, gt2,
        embx_c[:, :D], W2_c[:D], W2_c[D:], b2_c.reshape(1, D),
        Wc1, bc1.reshape(1, D), wc2p, bc2p,
        embx_a[:, :D], W2_a[:D], W2_a[D:], b2_a.reshape(1, D),
        Wa1, ba1.reshape(1, D), Wa2, ba2.reshape(1, NUM_OUT))

    return (logits_p, value_p[:, :1])
